# baseline (speedup 1.0000x reference)
import functools

import jax
import jax.numpy as jnp
from jax import lax
from jax.experimental import pallas as pl
from jax.experimental.pallas import tpu as pltpu
from jax.experimental.pallas import tpu_sc as plsc

NUM_USERS = 1000000
NUM_JOKES = 100
EMBED_DIM = 16
BATCH = 16384

NC = 2
NS = 16
NW = NC * NS
B_PER_W = BATCH // NW
CHUNK = 128
N_CHUNKS = B_PER_W // CHUNK


def _sc_gather_body(uidx_hbm, jidx_hbm, utabT_hbm, jtab_hbm,
                    uoutT_hbm, jout_hbm,
                    uidx_v, jidx_v, urowsT_v, jrows_v, usem, jsem):
  wid = lax.axis_index("s") * NC + lax.axis_index("c")
  base = wid * B_PER_W
  pltpu.sync_copy(uidx_hbm.at[pl.ds(base, B_PER_W)], uidx_v)
  pltpu.sync_copy(jidx_hbm.at[pl.ds(base, B_PER_W)], jidx_v)
  copies = []
  for j in range(N_CHUNKS):
    sl = pl.ds(j * CHUNK, CHUNK)
    copies.append(pltpu.async_copy(
        jtab_hbm.at[jidx_v.at[sl]], jrows_v.at[sl], jsem))
    for s in range(EMBED_DIM):
      copies.append(pltpu.async_copy(
          utabT_hbm.at[s].at[uidx_v.at[sl]], urowsT_v.at[s, sl], usem))
  for c in copies:
    c.wait()
  pltpu.sync_copy(urowsT_v, uoutT_hbm.at[:, pl.ds(base, B_PER_W)])
  pltpu.sync_copy(jrows_v, jout_hbm.at[pl.ds(base, B_PER_W)])


_sc_gather = functools.partial(
    pl.kernel,
    out_type=(
        jax.ShapeDtypeStruct((EMBED_DIM, BATCH), jnp.float32),
        jax.ShapeDtypeStruct((BATCH, EMBED_DIM), jnp.float32),
    ),
    mesh=plsc.VectorSubcoreMesh(
        core_axis_name="c", subcore_axis_name="s",
        num_cores=NC, num_subcores=NS),
    compiler_params=pltpu.CompilerParams(use_tc_tiling_on_sc=False),
    scratch_types=[
        pltpu.VMEM((B_PER_W,), jnp.int32),
        pltpu.VMEM((B_PER_W,), jnp.int32),
        pltpu.VMEM((EMBED_DIM, B_PER_W), jnp.float32),
        pltpu.VMEM((B_PER_W, EMBED_DIM), jnp.float32),
        pltpu.SemaphoreType.DMA,
        pltpu.SemaphoreType.DMA,
    ],
)(_sc_gather_body)


def _mlp_body(uT_ref, j_ref, w1u_ref, w1j_ref, b1_ref, w2_ref, b2_ref,
              w3_ref, b3_ref, o_ref):
  dotT = functools.partial(
      lax.dot_general, dimension_numbers=(((0,), (0,)), ((), ())),
      preferred_element_type=jnp.float32)
  dot = functools.partial(jnp.dot, preferred_element_type=jnp.float32)
  h1 = dotT(uT_ref[...], w1u_ref[...]) + dot(j_ref[...], w1j_ref[...])
  h1 = jnp.maximum(h1 + b1_ref[...], 0.0)
  h2 = jnp.maximum(dot(h1, w2_ref[...]) + b2_ref[...], 0.0)
  y = dot(h2, w3_ref[...]) + b3_ref[...]
  o_ref[...] = jnp.tanh(y) * 10.0


def _mlp(u_embT, j_emb, W1u, W1j, b1, W2, b2, W3, b3):
  blk = 2048
  grid = (BATCH // blk,)
  rep = lambda i: (0, 0)
  return pl.pallas_call(
      _mlp_body,
      grid=grid,
      in_specs=[
          pl.BlockSpec((EMBED_DIM, blk), lambda i: (0, i)),
          pl.BlockSpec((blk, EMBED_DIM), lambda i: (i, 0)),
          pl.BlockSpec((EMBED_DIM, 128), rep),
          pl.BlockSpec((EMBED_DIM, 128), rep),
          pl.BlockSpec((1, 128), rep),
          pl.BlockSpec((128, 64), rep),
          pl.BlockSpec((1, 64), rep),
          pl.BlockSpec((64, 1), rep),
          pl.BlockSpec((1, 1), rep),
      ],
      out_specs=pl.BlockSpec((blk, 1), lambda i: (i, 0)),
      out_shape=jax.ShapeDtypeStruct((BATCH, 1), jnp.float32),
  )(u_embT, j_emb, W1u, W1j, b1, W2, b2, W3, b3)


def kernel(user, joke, user_table, joke_table, W1, b1, W2, b2, W3, b3):
  user = user.astype(jnp.int32)
  joke = joke.astype(jnp.int32)
  u_embT, j_emb = _sc_gather(user, joke, user_table.T, joke_table)
  W1u = W1[:EMBED_DIM]
  W1j = W1[EMBED_DIM:]
  return _mlp(u_embT, j_emb, W1u, W1j,
              b1.reshape(1, 128), W2, b2.reshape(1, 64),
              W3, b3.reshape(1, 1))


# speedup vs baseline: 11.5549x; 11.5549x over previous
import functools

import jax
import jax.numpy as jnp
from jax import lax
from jax.experimental import pallas as pl
from jax.experimental.pallas import tpu as pltpu
from jax.experimental.pallas import tpu_sc as plsc

NUM_USERS = 1000000
NUM_JOKES = 100
EMBED_DIM = 16
BATCH = 16384

NC = 2
NS = 16
NW = NC * NS
B_PER_W = BATCH // NW
GROUP = 16
N_GROUPS = B_PER_W // GROUP


def _sc_gather_body(uidx_hbm, jidx_hbm, utabT_hbm, jtabT_hbm,
                    uout_hbm, jout_hbm,
                    uidx_v, jidx_v, jtab_v, ublk_v,
                    urows_v, jrows_v, bsem, osem):
  wid = lax.axis_index("s") * NC + lax.axis_index("c")
  base = wid * B_PER_W

  pltpu.sync_copy(uidx_hbm.at[pl.ds(base, B_PER_W)], uidx_v)
  pltpu.sync_copy(jidx_hbm.at[pl.ds(base, B_PER_W)], jidx_v)
  pltpu.sync_copy(jtabT_hbm, jtab_v)

  rows16 = lax.iota(jnp.int32, 16)

  def group_body(g, _):
    uvec_g = uidx_v[pl.ds(g * GROUP, GROUP)]
    jvec_g = jidx_v[pl.ds(g * GROUP, GROUP)]
    copies = []
    for k in range(GROUP):
      u = uvec_g[k]
      l0 = pl.multiple_of((u >> 7) << 7, 128)
      copies.append(pltpu.async_copy(
          utabT_hbm.at[:, pl.ds(l0, 128)], ublk_v.at[k], bsem))
    for k in range(GROUP):
      b = g * GROUP + k
      j = jvec_g[k]
      jvec = plsc.load_gather(jtab_v, [rows16, jnp.full((16,), j, jnp.int32)])
      plsc.store_scatter(jrows_v, [rows16, jnp.full((16,), b, jnp.int32)],
                         jvec)
    for c in copies:
      c.wait()
    for k in range(GROUP):
      b = g * GROUP + k
      lane = jnp.full((16,), uvec_g[k] & 127, jnp.int32)
      uvec = plsc.load_gather(ublk_v.at[k], [rows16, lane])
      plsc.store_scatter(urows_v, [rows16, jnp.full((16,), b, jnp.int32)],
                         uvec)
    return 0

  lax.fori_loop(0, N_GROUPS, group_body, 0)

  pltpu.async_copy(urows_v, uout_hbm.at[:, pl.ds(base, B_PER_W)], osem).wait()
  pltpu.async_copy(jrows_v, jout_hbm.at[:, pl.ds(base, B_PER_W)], osem).wait()


_sc_gather = functools.partial(
    pl.kernel,
    out_type=(
        jax.ShapeDtypeStruct((EMBED_DIM, BATCH), jnp.float32),
        jax.ShapeDtypeStruct((EMBED_DIM, BATCH), jnp.float32),
    ),
    mesh=plsc.VectorSubcoreMesh(
        core_axis_name="c", subcore_axis_name="s",
        num_cores=NC, num_subcores=NS),
    compiler_params=pltpu.CompilerParams(use_tc_tiling_on_sc=True,
                                         needs_layout_passes=False),
    scratch_types=[
        pltpu.VMEM((B_PER_W,), jnp.int32),
        pltpu.VMEM((B_PER_W,), jnp.int32),
        pltpu.VMEM((EMBED_DIM, 128), jnp.float32),
        pltpu.VMEM((GROUP, EMBED_DIM, 128), jnp.float32),
        pltpu.VMEM((EMBED_DIM, B_PER_W), jnp.float32),
        pltpu.VMEM((EMBED_DIM, B_PER_W), jnp.float32),
        pltpu.SemaphoreType.DMA,
        pltpu.SemaphoreType.DMA,
    ],
)(_sc_gather_body)


def _mlp_body(uT_ref, jT_ref, w1u_ref, w1j_ref, b1_ref, w2_ref, b2_ref,
              w3_ref, b3_ref, o_ref):
  dotT = functools.partial(
      lax.dot_general, dimension_numbers=(((0,), (0,)), ((), ())),
      preferred_element_type=jnp.float32)
  dot = functools.partial(jnp.dot, preferred_element_type=jnp.float32)
  h1 = dotT(uT_ref[...], w1u_ref[...]) + dotT(jT_ref[...], w1j_ref[...])
  h1 = jnp.maximum(h1 + b1_ref[...], 0.0)
  h2 = jnp.maximum(dot(h1, w2_ref[...]) + b2_ref[...], 0.0)
  y = dot(h2, w3_ref[...]) + b3_ref[...]
  o_ref[...] = jnp.tanh(y) * 10.0


def _mlp(u_emb, j_emb, W1u, W1j, b1, W2, b2, W3, b3):
  blk = 2048
  grid = (BATCH // blk,)
  rep = lambda i: (0, 0)
  return pl.pallas_call(
      _mlp_body,
      grid=grid,
      in_specs=[
          pl.BlockSpec((EMBED_DIM, blk), lambda i: (0, i)),
          pl.BlockSpec((EMBED_DIM, blk), lambda i: (0, i)),
          pl.BlockSpec((EMBED_DIM, 128), rep),
          pl.BlockSpec((EMBED_DIM, 128), rep),
          pl.BlockSpec((1, 128), rep),
          pl.BlockSpec((128, 64), rep),
          pl.BlockSpec((1, 64), rep),
          pl.BlockSpec((64, 1), rep),
          pl.BlockSpec((1, 1), rep),
      ],
      out_specs=pl.BlockSpec((blk, 1), lambda i: (i, 0)),
      out_shape=jax.ShapeDtypeStruct((BATCH, 1), jnp.float32),
  )(u_emb, j_emb, W1u, W1j, b1, W2, b2, W3, b3)


def kernel(user, joke, user_table, joke_table, W1, b1, W2, b2, W3, b3):
  user = user.astype(jnp.int32)
  joke = joke.astype(jnp.int32)
  jtabT = jnp.pad(joke_table.T, ((0, 0), (0, 128 - NUM_JOKES)))
  u_emb, j_emb = _sc_gather(user, joke, user_table.T, jtabT)
  W1u = W1[:EMBED_DIM]
  W1j = W1[EMBED_DIM:]
  return _mlp(u_emb, j_emb, W1u, W1j,
              b1.reshape(1, 128), W2, b2.reshape(1, 64),
              W3, b3.reshape(1, 1))


# speedup vs baseline: 13.3293x; 1.1536x over previous
import functools

import jax
import jax.numpy as jnp
from jax import lax
from jax.experimental import pallas as pl
from jax.experimental.pallas import tpu as pltpu
from jax.experimental.pallas import tpu_sc as plsc

NUM_USERS = 1000000
NUM_JOKES = 100
EMBED_DIM = 16
BATCH = 16384

NC = 2
NS = 16
NW = NC * NS
B_PER_W = BATCH // NW
GROUP = 16
N_GROUPS = B_PER_W // GROUP


def _sc_gather_body(uidx_hbm, jidx_hbm, utabT_hbm, jtabT_hbm,
                    uout_hbm, jout_hbm,
                    uidx_v, jidx_v, jtab_v, ublk_v,
                    urows_v, jrows_v, bsem, bsem1, osem):
  wid = lax.axis_index("s") * NC + lax.axis_index("c")
  base = wid * B_PER_W

  pltpu.sync_copy(uidx_hbm.at[pl.ds(base, B_PER_W)], uidx_v)
  pltpu.sync_copy(jidx_hbm.at[pl.ds(base, B_PER_W)], jidx_v)
  pltpu.sync_copy(jtabT_hbm, jtab_v)

  rows16 = lax.iota(jnp.int32, 16)

  def fire(g, buf, sem):
    uvec_g = uidx_v[pl.ds(g * GROUP, GROUP)]
    for k in range(GROUP):
      l0 = pl.multiple_of((uvec_g[k] >> 7) << 7, 128)
      pltpu.async_copy(utabT_hbm.at[:, pl.ds(l0, 128)],
                       ublk_v.at[buf, k], sem)

  def drain(buf, sem):
    for k in range(GROUP):
      pltpu.make_async_copy(utabT_hbm.at[:, pl.ds(0, 128)],
                            ublk_v.at[buf, k], sem).wait()

  def extract_jokes(g):
    jvec_g = jidx_v[pl.ds(g * GROUP, GROUP)]
    for k in range(GROUP):
      b = g * GROUP + k
      jvec = plsc.load_gather(
          jtab_v, [rows16, jnp.full((16,), jvec_g[k], jnp.int32)])
      plsc.store_scatter(jrows_v, [rows16, jnp.full((16,), b, jnp.int32)],
                         jvec)

  def extract_users(g, buf):
    uvec_g = uidx_v[pl.ds(g * GROUP, GROUP)]
    for k in range(GROUP):
      b = g * GROUP + k
      lane = jnp.full((16,), uvec_g[k] & 127, jnp.int32)
      uvec = plsc.load_gather(ublk_v.at[buf, k], [rows16, lane])
      plsc.store_scatter(urows_v, [rows16, jnp.full((16,), b, jnp.int32)],
                         uvec)

  fire(0, 0, bsem)

  def pair_body(i, _):
    g0 = 2 * i
    fire(g0 + 1, 1, bsem1)
    extract_jokes(g0)
    drain(0, bsem)
    extract_users(g0, 0)

    @pl.when(g0 + 2 < N_GROUPS)
    def _():
      fire(g0 + 2, 0, bsem)

    extract_jokes(g0 + 1)
    drain(1, bsem1)
    extract_users(g0 + 1, 1)
    return 0

  lax.fori_loop(0, N_GROUPS // 2, pair_body, 0)

  pltpu.async_copy(urows_v, uout_hbm.at[:, pl.ds(base, B_PER_W)], osem).wait()
  pltpu.async_copy(jrows_v, jout_hbm.at[:, pl.ds(base, B_PER_W)], osem).wait()


_sc_gather = functools.partial(
    pl.kernel,
    out_type=(
        jax.ShapeDtypeStruct((EMBED_DIM, BATCH), jnp.float32),
        jax.ShapeDtypeStruct((EMBED_DIM, BATCH), jnp.float32),
    ),
    mesh=plsc.VectorSubcoreMesh(
        core_axis_name="c", subcore_axis_name="s",
        num_cores=NC, num_subcores=NS),
    compiler_params=pltpu.CompilerParams(use_tc_tiling_on_sc=True,
                                         needs_layout_passes=False),
    scratch_types=[
        pltpu.VMEM((B_PER_W,), jnp.int32),
        pltpu.VMEM((B_PER_W,), jnp.int32),
        pltpu.VMEM((EMBED_DIM, 128), jnp.float32),
        pltpu.VMEM((2, GROUP, EMBED_DIM, 128), jnp.float32),
        pltpu.VMEM((EMBED_DIM, B_PER_W), jnp.float32),
        pltpu.VMEM((EMBED_DIM, B_PER_W), jnp.float32),
        pltpu.SemaphoreType.DMA,
        pltpu.SemaphoreType.DMA,
        pltpu.SemaphoreType.DMA,
    ],
)(_sc_gather_body)


def _mlp_body(uT_ref, jT_ref, w1u_ref, w1j_ref, b1_ref, w2_ref, b2_ref,
              w3_ref, b3_ref, o_ref):
  dotT = functools.partial(
      lax.dot_general, dimension_numbers=(((0,), (0,)), ((), ())),
      preferred_element_type=jnp.float32)
  dot = functools.partial(jnp.dot, preferred_element_type=jnp.float32)
  h1 = dotT(uT_ref[...], w1u_ref[...]) + dotT(jT_ref[...], w1j_ref[...])
  h1 = jnp.maximum(h1 + b1_ref[...], 0.0)
  h2 = jnp.maximum(dot(h1, w2_ref[...]) + b2_ref[...], 0.0)
  y = dot(h2, w3_ref[...]) + b3_ref[...]
  o_ref[...] = jnp.tanh(y) * 10.0


def _mlp(u_emb, j_emb, W1u, W1j, b1, W2, b2, W3, b3):
  blk = 2048
  grid = (BATCH // blk,)
  rep = lambda i: (0, 0)
  return pl.pallas_call(
      _mlp_body,
      grid=grid,
      in_specs=[
          pl.BlockSpec((EMBED_DIM, blk), lambda i: (0, i)),
          pl.BlockSpec((EMBED_DIM, blk), lambda i: (0, i)),
          pl.BlockSpec((EMBED_DIM, 128), rep),
          pl.BlockSpec((EMBED_DIM, 128), rep),
          pl.BlockSpec((1, 128), rep),
          pl.BlockSpec((128, 64), rep),
          pl.BlockSpec((1, 64), rep),
          pl.BlockSpec((64, 1), rep),
          pl.BlockSpec((1, 1), rep),
      ],
      out_specs=pl.BlockSpec((blk, 1), lambda i: (i, 0)),
      out_shape=jax.ShapeDtypeStruct((BATCH, 1), jnp.float32),
  )(u_emb, j_emb, W1u, W1j, b1, W2, b2, W3, b3)


def kernel(user, joke, user_table, joke_table, W1, b1, W2, b2, W3, b3):
  user = user.astype(jnp.int32)
  joke = joke.astype(jnp.int32)
  jtabT = jnp.pad(joke_table.T, ((0, 0), (0, 128 - NUM_JOKES)))
  u_emb, j_emb = _sc_gather(user, joke, user_table.T, jtabT)
  W1u = W1[:EMBED_DIM]
  W1j = W1[EMBED_DIM:]
  return _mlp(u_emb, j_emb, W1u, W1j,
              b1.reshape(1, 128), W2, b2.reshape(1, 64),
              W3, b3.reshape(1, 1))
